# baseline (speedup 1.0000x reference)
import jax
import jax.numpy as jnp
from jax.experimental import pallas as pl
from jax.experimental.pallas import tpu as pltpu

_TM = 256
_TK = 1024
_CR = 1024
_NC = 4
_NCLEAN = 12
_NW = 4


def _ffn_kernel(x_ref, x_hbm, w1_hbm, b1_ref, w2_hbm, b2_ref, o_ref,
                xc, w1v, w2v, acc, sem_x, sem1, sem2):
    g = pl.program_id(0)
    d_model = x_hbm.shape[1]
    d_ff = w1v.shape[1]
    m_all = x_hbm.shape[0]

    def xc_copy():
        return pltpu.make_async_copy(
            x_hbm.at[pl.ds(m_all - _CR, _CR), :], xc, sem_x)

    def w1_copy(c):
        return pltpu.make_async_copy(
            w1_hbm.at[:, pl.ds(c * _TK, _TK)],
            w1v.at[:, pl.ds(c * _TK, _TK)], sem1.at[c])

    def w2_copy(c):
        return pltpu.make_async_copy(
            w2_hbm.at[pl.ds(c * _TK, _TK), :],
            w2v.at[pl.ds(c * _TK, _TK), :], sem2.at[c])

    @pl.when(g == 0)
    def _():
        xc_copy().start()
        for c in range(_NC):
            w1_copy(c).start()
            w2_copy(c).start()

    @pl.when(g < _NC)
    def _chunked():
        @pl.when(g == 0)
        def _():
            xc_copy().wait()
        w1_copy(g).wait()
        w2_copy(g).wait()
        hc = jnp.dot(xc[...], w1v[:, pl.ds(g * _TK, _TK)],
                     preferred_element_type=jnp.float32)
        hc = jnp.maximum(hc + b1_ref[:, pl.ds(g * _TK, _TK)], 0.0)
        contrib = jnp.dot(hc, w2v[pl.ds(g * _TK, _TK), :],
                          preferred_element_type=jnp.float32)

        @pl.when(g == 0)
        def _():
            acc[...] = contrib + b2_ref[...]

        @pl.when(g > 0)
        def _():
            acc[...] += contrib

    @pl.when((g >= _NC) & (g < _NC + _NCLEAN))
    def _clean():
        h = jnp.dot(x_ref[...], w1v[...], preferred_element_type=jnp.float32)
        h = jnp.maximum(h + b1_ref[...], 0.0)
        out = jnp.dot(h, w2v[...], preferred_element_type=jnp.float32)
        o_ref[...] = out + b2_ref[...]

    @pl.when(g >= _NC + _NCLEAN)
    def _writeout():
        o_ref[...] = acc[pl.ds((g - _NC - _NCLEAN) * _TM, _TM), :]


def _x_index(g):
    t = jnp.clip(g - _NC, 0, _NCLEAN - 1)
    return (t, 0)


def _o_index(g):
    t = jnp.maximum(g - _NC, 0)
    return (t, 0)


def kernel(x, w1, b1, w2, b2):
    B, S, d_model = x.shape
    d_ff = w1.shape[1]
    M = B * S

    x2d = x.reshape(M, d_model)
    b1_2d = b1.reshape(1, d_ff)
    b2_2d = b2.reshape(1, d_model)

    out2d = pl.pallas_call(
        _ffn_kernel,
        out_shape=jax.ShapeDtypeStruct((M, d_model), jnp.float32),
        grid=(_NC + _NCLEAN + _NW,),
        in_specs=[
            pl.BlockSpec((_TM, d_model), _x_index),
            pl.BlockSpec(memory_space=pltpu.MemorySpace.HBM),
            pl.BlockSpec(memory_space=pltpu.MemorySpace.HBM),
            pl.BlockSpec((1, d_ff), lambda g: (0, 0)),
            pl.BlockSpec(memory_space=pltpu.MemorySpace.HBM),
            pl.BlockSpec((1, d_model), lambda g: (0, 0)),
        ],
        out_specs=pl.BlockSpec((_TM, d_model), _o_index),
        scratch_shapes=[
            pltpu.VMEM((_CR, d_model), jnp.float32),
            pltpu.VMEM((d_model, d_ff), jnp.float32),
            pltpu.VMEM((d_ff, d_model), jnp.float32),
            pltpu.VMEM((_CR, d_model), jnp.float32),
            pltpu.SemaphoreType.DMA,
            pltpu.SemaphoreType.DMA((_NC,)),
            pltpu.SemaphoreType.DMA((_NC,)),
        ],
        compiler_params=pltpu.CompilerParams(
            dimension_semantics=("arbitrary",),
            vmem_limit_bytes=60 * 1024 * 1024,
        ),
        cost_estimate=pl.CostEstimate(
            flops=4 * M * d_model * d_ff,
            transcendentals=0,
            bytes_accessed=(x2d.size + _CR * d_model + w1.size + b1.size
                            + w2.size + b2.size + M * d_model) * 4,
        ),
    )(x2d, x2d, w1, b1_2d, w2, b2_2d)

    return out2d.reshape(B, S, d_model)


# speedup vs baseline: 1.1226x; 1.1226x over previous
import jax
import jax.numpy as jnp
from jax.experimental import pallas as pl
from jax.experimental.pallas import tpu as pltpu

_TM = 512
_NQ = 4


def _ffn_kernel(x_ref, w1_hbm, b1_ref, w2_hbm, b2_ref, o_ref,
                w1v, w2v, sem1, sem2):
    i = pl.program_id(0)
    d_model = w1v.shape[0]
    d_ff = w2v.shape[0]
    q1 = d_model // _NQ
    q2 = d_ff // _NQ

    def w1_copy(q):
        return pltpu.make_async_copy(
            w1_hbm.at[pl.ds(q * q1, q1), :],
            w1v.at[pl.ds(q * q1, q1), :], sem1.at[q])

    def w2_copy(q):
        return pltpu.make_async_copy(
            w2_hbm.at[pl.ds(q * q2, q2), :],
            w2v.at[pl.ds(q * q2, q2), :], sem2.at[q])

    @pl.when(i == 0)
    def _first():
        for q in range(_NQ):
            w1_copy(q).start()
        for q in range(_NQ):
            w2_copy(q).start()
        x_val = x_ref[...]
        h = None
        for q in range(_NQ):
            w1_copy(q).wait()
            p = jnp.dot(x_val[:, q * q1:(q + 1) * q1],
                        w1v[pl.ds(q * q1, q1), :],
                        preferred_element_type=jnp.float32)
            h = p if h is None else h + p
        h = jnp.maximum(h + b1_ref[...], 0.0)
        out = None
        for q in range(_NQ):
            w2_copy(q).wait()
            p = jnp.dot(h[:, q * q2:(q + 1) * q2],
                        w2v[pl.ds(q * q2, q2), :],
                        preferred_element_type=jnp.float32)
            out = p if out is None else out + p
        o_ref[...] = out + b2_ref[...]

    @pl.when(i > 0)
    def _rest():
        h = jnp.dot(x_ref[...], w1v[...], preferred_element_type=jnp.float32)
        h = jnp.maximum(h + b1_ref[...], 0.0)
        out = jnp.dot(h, w2v[...], preferred_element_type=jnp.float32)
        o_ref[...] = out + b2_ref[...]


def kernel(x, w1, b1, w2, b2):
    B, S, d_model = x.shape
    d_ff = w1.shape[1]
    M = B * S

    x2d = x.reshape(M, d_model)
    b1_2d = b1.reshape(1, d_ff)
    b2_2d = b2.reshape(1, d_model)

    out2d = pl.pallas_call(
        _ffn_kernel,
        out_shape=jax.ShapeDtypeStruct((M, d_model), jnp.float32),
        grid=(M // _TM,),
        in_specs=[
            pl.BlockSpec((_TM, d_model), lambda i: (i, 0)),
            pl.BlockSpec(memory_space=pltpu.MemorySpace.HBM),
            pl.BlockSpec((1, d_ff), lambda i: (0, 0)),
            pl.BlockSpec(memory_space=pltpu.MemorySpace.HBM),
            pl.BlockSpec((1, d_model), lambda i: (0, 0)),
        ],
        out_specs=pl.BlockSpec((_TM, d_model), lambda i: (i, 0)),
        scratch_shapes=[
            pltpu.VMEM((d_model, d_ff), jnp.float32),
            pltpu.VMEM((d_ff, d_model), jnp.float32),
            pltpu.SemaphoreType.DMA((_NQ,)),
            pltpu.SemaphoreType.DMA((_NQ,)),
        ],
        compiler_params=pltpu.CompilerParams(
            dimension_semantics=("arbitrary",),
            vmem_limit_bytes=60 * 1024 * 1024,
        ),
        cost_estimate=pl.CostEstimate(
            flops=4 * M * d_model * d_ff,
            transcendentals=0,
            bytes_accessed=(x2d.size + w1.size + b1.size + w2.size + b2.size
                            + M * d_model) * 4,
        ),
    )(x2d, w1, b1_2d, w2, b2_2d)

    return out2d.reshape(B, S, d_model)
